# baseline (speedup 1.0000x reference)
import jax
import jax.numpy as jnp
from jax.experimental import pallas as pl


_BLOCK_S = 256


def _copy_body(emb_ref, out_ref):
    blk = emb_ref[...]
    out_ref[...] = jnp.broadcast_to(blk[None, :, :], out_ref.shape)


def kernel(x, emb):
    batch, seq_len, d_model = x.shape
    grid = (seq_len // _BLOCK_S,)
    return pl.pallas_call(
        _copy_body,
        grid=grid,
        in_specs=[pl.BlockSpec((_BLOCK_S, d_model), lambda i: (i, 0))],
        out_specs=pl.BlockSpec((batch, _BLOCK_S, d_model), lambda i: (0, i, 0)),
        out_shape=jax.ShapeDtypeStruct((batch, seq_len, d_model), emb.dtype),
    )(emb[:seq_len])
